# baseline (speedup 1.0000x reference)
import functools

import jax
import jax.numpy as jnp
from jax import lax
from jax.experimental import pallas as pl
from jax.experimental.pallas import tpu as pltpu
from jax.experimental.pallas import tpu_sc as plsc

D_MODEL = 1024
NUM_CORES = 2
NUM_SUBCORES = 16
NUM_WORKERS = NUM_CORES * NUM_SUBCORES
CHUNKS = (48, 40, 32, 48, 40, 32, 16)
BUFROWS = (48, 40, 32)
NBUF = 3


def _emb_body(per_worker, w_per_row, ids_hbm, table_hbm, out_hbm,
              idx_v, buf0, buf1, buf2, gsem0, gsem1, gsem2,
              psem0, psem1, psem2):
    wid = lax.axis_index("s") * NUM_CORES + lax.axis_index("c")
    base = wid * per_worker
    row = wid // w_per_row
    col = (wid % w_per_row) * per_worker
    pltpu.sync_copy(ids_hbm.at[row, pl.ds(col, per_worker)], idx_v)

    bufs = (buf0, buf1, buf2)
    gsems = (gsem0, gsem1, gsem2)
    psems = (psem0, psem1, psem2)
    n_chunks = len(CHUNKS)
    offs = [sum(CHUNKS[:j]) for j in range(n_chunks)]
    gat = [None] * NBUF
    put = [None] * NBUF
    for j, c in enumerate(CHUNKS):
        b = j % NBUF
        if j >= NBUF:
            put[b].wait()
        gat[b] = pltpu.async_copy(
            table_hbm.at[idx_v.at[pl.ds(offs[j], c)]],
            bufs[b].at[pl.ds(0, c)], gsems[b])
        if j >= 1:
            pb = (j - 1) % NBUF
            pc = CHUNKS[j - 1]
            gat[pb].wait()
            put[pb] = pltpu.async_copy(
                bufs[pb].at[pl.ds(0, pc)],
                out_hbm.at[pl.ds(base + offs[j - 1], pc)], psems[pb])
    lb = (n_chunks - 1) % NBUF
    gat[lb].wait()
    put[lb] = pltpu.async_copy(
        bufs[lb].at[pl.ds(0, CHUNKS[-1])],
        out_hbm.at[pl.ds(base + offs[-1], CHUNKS[-1])], psems[lb])
    for b in range(NBUF):
        put[b].wait()


def kernel(input_ids, embedding_table):
    batch, seq = input_ids.shape
    n = batch * seq
    per_worker = n // NUM_WORKERS
    assert n % NUM_WORKERS == 0 and sum(CHUNKS) == per_worker
    assert seq % per_worker == 0
    w_per_row = seq // per_worker

    mesh = plsc.VectorSubcoreMesh(core_axis_name="c", subcore_axis_name="s")
    emb = pl.kernel(
        functools.partial(_emb_body, per_worker, w_per_row),
        mesh=mesh,
        out_type=jax.ShapeDtypeStruct((n, D_MODEL), jnp.float32),
        scratch_types=[
            pltpu.VMEM((per_worker,), jnp.int32),
        ] + [pltpu.VMEM((r, D_MODEL), jnp.float32) for r in BUFROWS]
          + [pltpu.SemaphoreType.DMA] * (2 * NBUF),
    )
    out = emb(input_ids, embedding_table)
    return out.reshape(batch, seq, D_MODEL)


# speedup vs baseline: 1.0448x; 1.0448x over previous
import functools

import jax
import jax.numpy as jnp
from jax import lax
from jax.experimental import pallas as pl
from jax.experimental.pallas import tpu as pltpu
from jax.experimental.pallas import tpu_sc as plsc

D_MODEL = 1024
NUM_CORES = 2
NUM_SUBCORES = 16
NUM_WORKERS = NUM_CORES * NUM_SUBCORES
CHUNKS = (56, 64, 56, 64, 16)
BUFROWS = (56, 64)
NBUF = 2


def _emb_body(per_worker, w_per_row, ids_hbm, table_hbm, out_hbm,
              idx_v, buf0, buf1, gsem0, gsem1, psem0, psem1):
    wid = lax.axis_index("s") * NUM_CORES + lax.axis_index("c")
    base = wid * per_worker
    row = wid // w_per_row
    col = (wid % w_per_row) * per_worker
    pltpu.sync_copy(ids_hbm.at[row, pl.ds(col, per_worker)], idx_v)

    bufs = (buf0, buf1)
    gsems = (gsem0, gsem1)
    psems = (psem0, psem1)
    n_chunks = len(CHUNKS)
    offs = [sum(CHUNKS[:j]) for j in range(n_chunks)]
    gat = [None] * NBUF
    put = [None] * NBUF
    for j, c in enumerate(CHUNKS):
        b = j % NBUF
        if j >= NBUF:
            put[b].wait()
        gat[b] = pltpu.async_copy(
            table_hbm.at[idx_v.at[pl.ds(offs[j], c)]],
            bufs[b].at[pl.ds(0, c)], gsems[b])
        if j >= 1:
            pb = (j - 1) % NBUF
            pc = CHUNKS[j - 1]
            gat[pb].wait()
            put[pb] = pltpu.async_copy(
                bufs[pb].at[pl.ds(0, pc)],
                out_hbm.at[pl.ds(base + offs[j - 1], pc)], psems[pb])
    lb = (n_chunks - 1) % NBUF
    gat[lb].wait()
    put[lb] = pltpu.async_copy(
        bufs[lb].at[pl.ds(0, CHUNKS[-1])],
        out_hbm.at[pl.ds(base + offs[-1], CHUNKS[-1])], psems[lb])
    for b in range(NBUF):
        put[b].wait()


def kernel(input_ids, embedding_table):
    batch, seq = input_ids.shape
    n = batch * seq
    per_worker = n // NUM_WORKERS
    assert n % NUM_WORKERS == 0 and sum(CHUNKS) == per_worker
    assert seq % per_worker == 0
    w_per_row = seq // per_worker

    mesh = plsc.VectorSubcoreMesh(core_axis_name="c", subcore_axis_name="s")
    emb = pl.kernel(
        functools.partial(_emb_body, per_worker, w_per_row),
        mesh=mesh,
        out_type=jax.ShapeDtypeStruct((n, D_MODEL), jnp.float32),
        scratch_types=[
            pltpu.VMEM((per_worker,), jnp.int32),
        ] + [pltpu.VMEM((r, D_MODEL), jnp.float32) for r in BUFROWS]
          + [pltpu.SemaphoreType.DMA] * (2 * NBUF),
    )
    out = emb(input_ids, embedding_table)
    return out.reshape(batch, seq, D_MODEL)
